# baseline (speedup 1.0000x reference)
import functools

import jax
import jax.numpy as jnp
from jax.experimental import pallas as pl
from jax.experimental.pallas import tpu as pltpu

N = 1024
D = 128
BK = 2048
RB = 128
NR = N // RB


def _nn_kernel(emb_ref, w_ref, b_ref, tab_ref, out_ref, best_ref, idx_ref,
               t2_ref, *, nv, blk):
    j = pl.program_id(0)
    i = pl.program_id(1)

    tb = tab_ref[...]

    @pl.when(i == 0)
    def _norms():
        ones_row = jnp.ones((1, D), jnp.float32)
        t2_ref[...] = 0.5 * jax.lax.dot_general(
            ones_row, tb * tb, (((1,), (1,)), ((), ())),
            precision=jax.lax.Precision.HIGHEST,
            preferred_element_type=jnp.float32)

    y = (emb_ref[...] - b_ref[...]) / (w_ref[...] + 1e-6)

    mm = jax.lax.dot_general(y, tb, (((1,), (1,)), ((), ())),
                             preferred_element_type=jnp.float32)
    s = t2_ref[...] - mm

    rows = pl.ds(i * RB, RB)
    col = j * blk + jax.lax.broadcasted_iota(jnp.int32, (1, blk), 1)
    colb = jnp.broadcast_to(col, (RB, blk))

    @pl.when(j == 0)
    def _init():
        best_ref[rows, :] = s
        idx_ref[rows, :] = colb

    @pl.when(j > 0)
    def _fold():
        prev = best_ref[rows, :]
        upd = s < prev
        best_ref[rows, :] = jnp.minimum(s, prev)
        idx_ref[rows, :] = jnp.where(upd, colb, idx_ref[rows, :])

    @pl.when(jnp.logical_and(j == nv - 1, i == NR - 1))
    def _done():
        m = best_ref[...]
        rowmin = jnp.min(m, axis=1, keepdims=True)
        big = jnp.int32(2147483647)
        cand = jnp.where(m == rowmin, idx_ref[...], big)
        out_ref[...] = jnp.min(cand, axis=1, keepdims=True)


@jax.jit
def kernel(embeddings, ln_weight, ln_bias, table):
    vocab = table.shape[0]
    nv = pl.cdiv(vocab, BK)
    padded = nv * BK
    if padded != vocab:
        table = jnp.pad(table, ((0, padded - vocab), (0, 0)), mode="edge")

    out = pl.pallas_call(
        functools.partial(_nn_kernel, nv=nv, blk=BK),
        grid=(nv, NR),
        in_specs=[
            pl.BlockSpec((RB, D), lambda j, i: (i, 0)),
            pl.BlockSpec((1, D), lambda j, i: (0, 0)),
            pl.BlockSpec((1, D), lambda j, i: (0, 0)),
            pl.BlockSpec((BK, D), lambda j, i: (j, 0)),
        ],
        out_specs=pl.BlockSpec((N, 1), lambda j, i: (0, 0)),
        out_shape=jax.ShapeDtypeStruct((N, 1), jnp.int32),
        scratch_shapes=[
            pltpu.VMEM((N, BK), jnp.float32),
            pltpu.VMEM((N, BK), jnp.int32),
            pltpu.VMEM((1, BK), jnp.float32),
        ],
    )(embeddings, ln_weight[None, :], ln_bias[None, :], table)
    return out[:, 0]


# speedup vs baseline: 2.3679x; 2.3679x over previous
import functools

import jax
import jax.numpy as jnp
from jax.experimental import pallas as pl
from jax.experimental.pallas import tpu as pltpu

N = 1024
D = 128
BK = 2048


def _nn_kernel(emb_ref, w_ref, b_ref, tab_ref, out_ref, best_ref, blk_ref,
               *, nsteps, blk):
    j = pl.program_id(0)

    tb = tab_ref[...]
    ones_row = jnp.ones((1, D), jnp.float32)
    contract = (((1,), (1,)), ((), ()))
    t2h = 0.5 * jax.lax.dot_general(ones_row, tb * tb, contract,
                                    precision=jax.lax.Precision.HIGHEST,
                                    preferred_element_type=jnp.float32)

    y = (emb_ref[...] - b_ref[...]) / (w_ref[...] + 1e-6)

    mm = jax.lax.dot_general(y, tb, contract,
                             preferred_element_type=jnp.float32)
    s = t2h - mm

    prev = jnp.where(j == 0, jnp.float32(jnp.inf), best_ref[...])
    upd = s < prev
    best_ref[...] = jnp.minimum(s, prev)
    blk_ref[...] = jnp.where(upd, j, blk_ref[...])

    @pl.when(j == nsteps - 1)
    def _done():
        m = best_ref[...]
        rowmin = jnp.min(m, axis=1, keepdims=True)
        lane = jax.lax.broadcasted_iota(jnp.int32, (1, blk), 1)
        gcol = blk_ref[...] * blk + lane
        big = jnp.int32(2147483647)
        cand = jnp.where(m == rowmin, gcol, big)
        out_ref[...] = jnp.min(cand, axis=1, keepdims=True)


@jax.jit
def kernel(embeddings, ln_weight, ln_bias, table):
    vocab = table.shape[0]
    nsteps = pl.cdiv(vocab, BK)
    padded = nsteps * BK
    if padded != vocab:
        table = jnp.pad(table, ((0, padded - vocab), (0, 0)), mode="edge")

    out = pl.pallas_call(
        functools.partial(_nn_kernel, nsteps=nsteps, blk=BK),
        grid=(nsteps,),
        in_specs=[
            pl.BlockSpec((N, D), lambda j: (0, 0)),
            pl.BlockSpec((1, D), lambda j: (0, 0)),
            pl.BlockSpec((1, D), lambda j: (0, 0)),
            pl.BlockSpec((BK, D), lambda j: (j, 0)),
        ],
        out_specs=pl.BlockSpec((N, 1), lambda j: (0, 0)),
        out_shape=jax.ShapeDtypeStruct((N, 1), jnp.int32),
        scratch_shapes=[
            pltpu.VMEM((N, BK), jnp.float32),
            pltpu.VMEM((N, BK), jnp.int32),
        ],
    )(embeddings, ln_weight[None, :], ln_bias[None, :], table)
    return out[:, 0]
